# Initial kernel scaffold; baseline (speedup 1.0000x reference)
import jax
import jax.numpy as jnp
from jax.experimental import pallas as pl


def kernel(x, edge_index, edge_type, lin1_w, lin1_b, lin2_w, lin2_b, W1, Wr1, a1, Wres1, rel1, W2, Wr2, a2, Wres2, rel2):
    raise NotImplementedError("write your pallas kernel here")


# speedup vs baseline: 9.0898x; 9.0898x over previous
import jax
import jax.numpy as jnp
from jax import lax
from jax.experimental import pallas as pl
from jax.experimental.pallas import tpu as pltpu
from jax.experimental.pallas import tpu_sc as plsc

N = 10000
D = 128
E = 320000
BETA = 0.05
NW = 32
EW = E // NW
EWP = 10240
CH = EWP // 128
NP = 10240
RB = 2000
GRID = N // RB


def _leaky(x, s):
    return jnp.where(x > 0, x, s * x)


def _dotT(a, b):
    return lax.dot_general(a, b, (((1,), (1,)), ((), ())),
                           preferred_element_type=jnp.float32)


def _tc1_body(x_ref, l1w_ref, l1b_ref, w_ref, a_ref, wr_ref, rel_ref,
              h0_ref, wx_ref, sd_ref, ss_ref, c_ref):
    i = pl.program_id(0)
    h0 = _leaky(_dotT(x_ref[...], l1w_ref[...]) + l1b_ref[...][None, :], 0.01)
    wx = _dotT(h0, w_ref[...])
    a = a_ref[...]
    h0_ref[...] = h0
    wx_ref[...] = wx
    sd_ref[...] = jnp.sum(wx * a[0, 0:D][None, :], axis=1)[None, None, :]
    ss_ref[...] = jnp.sum(wx * a[0, D:2 * D][None, :], axis=1)[None, None, :]

    @pl.when(i == 0)
    def _():
        rw = _dotT(rel_ref[...], wr_ref[...])
        c8 = jnp.sum(rw * a[0, 2 * D:3 * D][None, :], axis=1)
        c_ref[...] = jnp.pad(c8, (0, 120), constant_values=-1e30)[None, :]


def _tc2_body(u_ref, h0_ref, wres_ref, w2_ref, a2_ref, wr2_ref,
              rel2_ref, h1_ref, wx2_ref, sd2_ref, ss2_ref, c2_ref):
    i = pl.program_id(0)
    agg = u_ref[0] + u_ref[1]
    pre = agg + _dotT(h0_ref[...], wres_ref[...])
    h1 = jnp.where(pre > 0, pre, jnp.exp(jnp.minimum(pre, 0.0)) - 1.0)
    wx2 = _dotT(h1, w2_ref[...])
    a2 = a2_ref[...]
    h1_ref[...] = h1
    wx2_ref[...] = wx2
    sd2_ref[...] = jnp.sum(wx2 * a2[0, 0:D][None, :], axis=1)[None, None, :]
    ss2_ref[...] = jnp.sum(wx2 * a2[0, D:2 * D][None, :], axis=1)[None, None, :]

    @pl.when(i == 0)
    def _():
        rw = _dotT(rel2_ref[...], wr2_ref[...])
        c8 = jnp.sum(rw * a2[0, 2 * D:3 * D][None, :], axis=1)
        c2_ref[...] = jnp.pad(c8, (0, 120), constant_values=-1e30)[None, :]


def _tc3_body(u_ref, h1_ref, wres2_ref, l2w_ref, l2b_ref, y_ref):
    pre = (u_ref[0] + u_ref[1]) + _dotT(h1_ref[...], wres2_ref[...])
    t = jnp.where(pre > 0, pre, jnp.exp(jnp.minimum(pre, 0.0)) - 1.0)
    nrm = jnp.sqrt(jnp.sum(t * t, axis=1, keepdims=True))
    t = t / jnp.maximum(nrm, 1e-12)
    y_ref[...] = _leaky(_dotT(t, l2w_ref[...]) + l2b_ref[...][None, :], 0.01)


def _sca_body(sd_hbm, ss_hbm, c_hbm, dst_hbm, src_hbm, et_hbm,
              p_hbm, dpart_hbm,
              sd_v, ss_v, c_v, dst_v, src_v, et_v, p_v, zrow_v, dn_sh):
    cid = lax.axis_index("c")
    sid = lax.axis_index("s")
    wid = sid * 2 + cid
    pltpu.sync_copy(sd_hbm, sd_v)
    pltpu.sync_copy(ss_hbm, ss_v)
    pltpu.sync_copy(c_hbm, c_v)
    pltpu.sync_copy(dst_hbm.at[wid], dst_v)
    pltpu.sync_copy(src_hbm.at[wid], src_v)
    pltpu.sync_copy(et_hbm.at[wid], et_v)

    z16 = jnp.zeros((16,), jnp.float32)
    for g in range(8):
        zrow_v[pl.ds(g * 16, 16)] = z16

    def zbody(k, car):
        pltpu.sync_copy(zrow_v, dn_sh.at[pl.ds((sid * 5 + k) * 128, 128)])
        return car
    lax.fori_loop(0, 5, zbody, 0)
    plsc.subcore_barrier()

    def mbody(i, m):
        return jnp.maximum(m, ss_v[pl.ds(i * 16, 16)])
    m16 = lax.fori_loop(0, N // 16, mbody, jnp.full((16,), -1e30, jnp.float32))

    gdn = lax.GatherDimensionNumbers(offset_dims=(), collapsed_slice_dims=(0,),
                                     start_index_map=(0,))

    def bcast_max(v):
        iot = lax.iota(jnp.int32, 16)
        for sft in (8, 4, 2, 1):
            perm = jnp.bitwise_and(iot + sft, 15)
            pv = lax.gather(v, perm[:, None], gdn, slice_sizes=(1,),
                            mode=lax.GatherScatterMode.PROMISE_IN_BOUNDS)
            v = jnp.maximum(v, pv)
        return v
    kmax = bcast_max(m16) + bcast_max(c_v[...])

    lane = lax.iota(jnp.int32, 16)

    def chunk(j, car):
        for g in range(8):
            sl = pl.ds(g * 16, 16)
            d16 = dst_v[j, sl]
            s16 = src_v[j, sl]
            t16 = et_v[j, sl]
            sdg = plsc.load_gather(sd_v, [d16])
            ssg = plsc.load_gather(ss_v, [s16])
            cg = plsc.load_gather(c_v, [t16])
            z = sdg + ssg + cg
            e = jnp.where(z > 0, z, 0.2 * z)
            zm = sdg + kmax
            mub = jnp.where(zm > 0, zm, 0.2 * zm)
            p = jnp.exp(e - mub)
            valid = (j * 128 + g * 16 + lane) < EW
            p_v[j, sl] = jnp.where(valid, p, 0.0)
        return car
    lax.fori_loop(0, CH, chunk, 0)

    pltpu.sync_copy(p_v, p_hbm.at[wid])

    def scat(j, car):
        pltpu.sync_copy(p_v.at[j], dn_sh.at[dst_v.at[j]], add=True)
        return car
    lax.fori_loop(0, CH, scat, 0)
    plsc.subcore_barrier()

    @pl.when(sid == 0)
    def _():
        pltpu.sync_copy(dn_sh, dpart_hbm.at[cid])


def _zero_rows(rows_v):
    z16 = jnp.zeros((16,), jnp.float32)

    def zr(r, car):
        for g in range(4):
            rows_v[r, pl.ds(g * 16, 16)] = z16
        return car
    lax.fori_loop(0, 128, zr, 0)


def _scb_body(pm_hbm, pa_hbm, dm_hbm, da_hbm, coef_hbm, dst_hbm, src_hbm,
              wxl_hbm, wxr_hbm, u_hbm,
              pm_v, pa_v, dst_v, src_v, dpk_v, dnm_v, dna_v, coef_v, rows_v,
              agg_sh):
    cid = lax.axis_index("c")
    sid = lax.axis_index("s")
    wid = sid * 2 + cid
    pltpu.sync_copy(dst_hbm.at[wid], dst_v)
    pltpu.sync_copy(src_hbm.at[wid], src_v)
    pltpu.sync_copy(pm_hbm.at[wid], pm_v)
    pltpu.sync_copy(pa_hbm.at[wid], pa_v)
    pltpu.sync_copy(coef_hbm, coef_v)

    def _sum_denom(d_hbm, dn_v):
        pltpu.sync_copy(d_hbm.at[0], dn_v)
        pltpu.sync_copy(d_hbm.at[1], dpk_v)

        def sbody(i, car):
            sl = pl.ds(i * 16, 16)
            d = dn_v[sl] + dpk_v[sl] + 1e-16
            r = plsc.bitcast(jnp.full((16,), 0x7EF311C3, jnp.int32)
                             - plsc.bitcast(d, jnp.int32), jnp.float32)
            r = r * (2.0 - d * r)
            r = r * (2.0 - d * r)
            r = r * (2.0 - d * r)
            dn_v[sl] = r
            return car
        lax.fori_loop(0, NP // 16, sbody, 0)
    _sum_denom(dm_hbm, dnm_v)
    _sum_denom(da_hbm, dna_v)

    cm = coef_v[0, :]
    ca = coef_v[1, :]

    def wchunk(j, car):
        for g in range(8):
            sl = pl.ds(g * 16, 16)
            d16 = dst_v[j, sl]
            dnm_g = plsc.load_gather(dnm_v, [d16])
            dna_g = plsc.load_gather(dna_v, [d16])
            w = cm * pm_v[j, sl] * dnm_g + ca * pa_v[j, sl] * dna_g
            pm_v[j, sl] = w
        return car
    lax.fori_loop(0, CH, wchunk, 0)

    for half, wx_hbm in ((0, wxl_hbm), (1, wxr_hbm)):
        _zero_rows(rows_v)

        def za(k, car):
            pltpu.sync_copy(rows_v, agg_sh.at[pl.ds((sid * 5 + k) * 128, 128)])
            return car
        lax.fori_loop(0, 5, za, 0)
        plsc.subcore_barrier()

        def chunk(j, car):
            pltpu.sync_copy(wx_hbm.at[src_v.at[j]], rows_v)
            j16 = jnp.full((16,), j, jnp.int32)

            def row(r, car2):
                wb = plsc.load_gather(pm_v,
                                      [j16, jnp.full((16,), r, jnp.int32)])
                for g in range(4):
                    sl = pl.ds(g * 16, 16)
                    rows_v[r, sl] = rows_v[r, sl] * wb
                return car2
            lax.fori_loop(0, 128, row, 0)
            pltpu.sync_copy(rows_v, agg_sh.at[dst_v.at[j]], add=True)
            return car
        lax.fori_loop(0, CH, chunk, 0)
        plsc.subcore_barrier()
        rows = NP // 16
        pltpu.sync_copy(agg_sh.at[pl.ds(sid * rows, rows)],
                        u_hbm.at[half, cid, pl.ds(sid * rows, rows)])
        plsc.subcore_barrier()


def _full(shape):
    return pl.BlockSpec(shape, lambda i: tuple(0 for _ in shape))


def kernel(x, edge_index, edge_type, lin1_w, lin1_b, lin2_w, lin2_b,
           W1, Wr1, a1, Wres1, rel1, W2, Wr2, a2, Wres2, rel2):
    f32 = jnp.float32
    src = edge_index[0]
    dst = edge_index[1]

    def prep(v):
        v = v.reshape(NW, EW)
        v = jnp.pad(v, ((0, 0), (0, EWP - EW)))
        return v.reshape(NW, CH, 128)
    dstE = prep(dst)
    srcE = prep(src)
    etE = prep(edge_type)

    tc1 = pl.pallas_call(
        _tc1_body,
        grid=(GRID,),
        in_specs=[
            pl.BlockSpec((RB, D), lambda i: (i, 0)),
            _full((D, D)),
            _full((D,)),
            _full((D, D)),
            _full((1, 3 * D)),
            _full((D, 200)),
            _full((8, 200)),
        ],
        out_specs=[
            pl.BlockSpec((RB, D), lambda i: (i, 0)),
            pl.BlockSpec((RB, D), lambda i: (i, 0)),
            pl.BlockSpec((1, 1, RB), lambda i: (i, 0, 0)),
            pl.BlockSpec((1, 1, RB), lambda i: (i, 0, 0)),
            _full((1, 128)),
        ],
        out_shape=[
            jax.ShapeDtypeStruct((N, D), f32),
            jax.ShapeDtypeStruct((N, D), f32),
            jax.ShapeDtypeStruct((GRID, 1, RB), f32),
            jax.ShapeDtypeStruct((GRID, 1, RB), f32),
            jax.ShapeDtypeStruct((1, 128), f32),
        ],
    )
    h0, wx1, sd1, ss1, c1 = tc1(x, lin1_w, lin1_b, W1, a1, Wr1, rel1)

    mesh = plsc.VectorSubcoreMesh(core_axis_name="c", subcore_axis_name="s")

    sca = pl.kernel(
        _sca_body,
        out_type=(jax.ShapeDtypeStruct((NW, CH, 128), f32),
                  jax.ShapeDtypeStruct((2, NP), f32)),
        mesh=mesh,
        compiler_params=pltpu.CompilerParams(needs_layout_passes=False),
        scratch_types=[
            pltpu.VMEM((N,), f32),
            pltpu.VMEM((N,), f32),
            pltpu.VMEM((16,), f32),
            pltpu.VMEM((CH, 128), jnp.int32),
            pltpu.VMEM((CH, 128), jnp.int32),
            pltpu.VMEM((CH, 128), jnp.int32),
            pltpu.VMEM((CH, 128), f32),
            pltpu.VMEM((128,), f32),
            pltpu.VMEM_SHARED((NP,), f32),
        ],
    )
    p1, dpart1 = sca(sd1.reshape(N), ss1.reshape(N), c1[0, :16],
                     dstE, srcE, etE)

    scb = pl.kernel(
        _scb_body,
        out_type=jax.ShapeDtypeStruct((2, 2, NP, 64), f32),
        mesh=mesh,
        compiler_params=pltpu.CompilerParams(needs_layout_passes=False,
                                             use_tc_tiling_on_sc=False),
        scratch_types=[
            pltpu.VMEM((CH, 128), f32),
            pltpu.VMEM((CH, 128), f32),
            pltpu.VMEM((CH, 128), jnp.int32),
            pltpu.VMEM((CH, 128), jnp.int32),
            pltpu.VMEM((NP,), f32),
            pltpu.VMEM((NP,), f32),
            pltpu.VMEM((NP,), f32),
            pltpu.VMEM((2, 16), f32),
            pltpu.VMEM((128, 64), f32),
            pltpu.VMEM_SHARED((NP, 64), f32),
        ],
    )

    def run_scb(pm, pa, dm, da, coef, wx):
        uh = scb(pm, pa, dm, da, coef, dstE, srcE, wx[:, :64], wx[:, 64:])
        return jnp.concatenate([uh[0], uh[1]], axis=-1)[:, :N]

    coef1 = jnp.stack([jnp.full((16,), 1.0, f32), jnp.zeros((16,), f32)])
    u1 = run_scb(p1, p1, dpart1, dpart1, coef1, wx1)

    tc2 = pl.pallas_call(
        _tc2_body,
        grid=(GRID,),
        in_specs=[
            pl.BlockSpec((2, RB, D), lambda i: (0, i, 0)),
            pl.BlockSpec((RB, D), lambda i: (i, 0)),
            _full((D, D)),
            _full((D, D)),
            _full((1, 3 * D)),
            _full((D, 200)),
            _full((8, 200)),
        ],
        out_specs=[
            pl.BlockSpec((RB, D), lambda i: (i, 0)),
            pl.BlockSpec((RB, D), lambda i: (i, 0)),
            pl.BlockSpec((1, 1, RB), lambda i: (i, 0, 0)),
            pl.BlockSpec((1, 1, RB), lambda i: (i, 0, 0)),
            _full((1, 128)),
        ],
        out_shape=[
            jax.ShapeDtypeStruct((N, D), f32),
            jax.ShapeDtypeStruct((N, D), f32),
            jax.ShapeDtypeStruct((GRID, 1, RB), f32),
            jax.ShapeDtypeStruct((GRID, 1, RB), f32),
            jax.ShapeDtypeStruct((1, 128), f32),
        ],
    )
    h1, wx2, sd2, ss2, c2 = tc2(u1, h0, Wres1, W2, a2, Wr2, rel2)

    p2, dpart2 = sca(sd2.reshape(N), ss2.reshape(N), c2[0, :16],
                     dstE, srcE, etE)

    coef2 = jnp.stack([jnp.full((16,), 1.0 - BETA, f32),
                       jnp.full((16,), BETA, f32)])
    u2 = run_scb(p2, p1, dpart2, dpart1, coef2, wx2)

    tc3 = pl.pallas_call(
        _tc3_body,
        grid=(GRID,),
        in_specs=[
            pl.BlockSpec((2, RB, D), lambda i: (0, i, 0)),
            pl.BlockSpec((RB, D), lambda i: (i, 0)),
            _full((D, D)),
            _full((D, D)),
            _full((D,)),
        ],
        out_specs=pl.BlockSpec((RB, D), lambda i: (i, 0)),
        out_shape=jax.ShapeDtypeStruct((N, D), f32),
    )
    return tc3(u2, h1, Wres2, lin2_w, lin2_b)


# speedup vs baseline: 10.2728x; 1.1301x over previous
import jax
import jax.numpy as jnp
from jax import lax
from jax.experimental import pallas as pl
from jax.experimental.pallas import tpu as pltpu
from jax.experimental.pallas import tpu_sc as plsc

N = 10000
D = 128
E = 320000
BETA = 0.05
NW = 32
EW = E // NW
EWP = 10240
CH = EWP // 128
NP = 10240
RB = 2000
GRID = N // RB


def _leaky(x, s):
    return jnp.where(x > 0, x, s * x)


def _dotT(a, b):
    return lax.dot_general(a, b, (((1,), (1,)), ((), ())),
                           preferred_element_type=jnp.float32)


def _tc1_body(x_ref, l1w_ref, l1b_ref, w_ref, a_ref, wr_ref, rel_ref,
              h0_ref, wx_ref, sd_ref, ss_ref, c_ref):
    i = pl.program_id(0)
    h0 = _leaky(_dotT(x_ref[...], l1w_ref[...]) + l1b_ref[...][None, :], 0.01)
    wx = _dotT(h0, w_ref[...])
    a = a_ref[...]
    h0_ref[...] = h0
    wx_ref[...] = wx
    sd_ref[...] = jnp.sum(wx * a[0, 0:D][None, :], axis=1)[None, None, :]
    ss_ref[...] = jnp.sum(wx * a[0, D:2 * D][None, :], axis=1)[None, None, :]

    @pl.when(i == 0)
    def _():
        rw = _dotT(rel_ref[...], wr_ref[...])
        c8 = jnp.sum(rw * a[0, 2 * D:3 * D][None, :], axis=1)
        c_ref[...] = jnp.pad(c8, (0, 120), constant_values=-1e30)[None, :]


def _tc2_body(u_ref, h0_ref, wres_ref, w2_ref, a2_ref, wr2_ref,
              rel2_ref, h1_ref, wx2_ref, sd2_ref, ss2_ref, c2_ref):
    i = pl.program_id(0)
    agg = u_ref[0] + u_ref[1]
    pre = agg + _dotT(h0_ref[...], wres_ref[...])
    h1 = jnp.where(pre > 0, pre, jnp.exp(jnp.minimum(pre, 0.0)) - 1.0)
    wx2 = _dotT(h1, w2_ref[...])
    a2 = a2_ref[...]
    h1_ref[...] = h1
    wx2_ref[...] = wx2
    sd2_ref[...] = jnp.sum(wx2 * a2[0, 0:D][None, :], axis=1)[None, None, :]
    ss2_ref[...] = jnp.sum(wx2 * a2[0, D:2 * D][None, :], axis=1)[None, None, :]

    @pl.when(i == 0)
    def _():
        rw = _dotT(rel2_ref[...], wr2_ref[...])
        c8 = jnp.sum(rw * a2[0, 2 * D:3 * D][None, :], axis=1)
        c2_ref[...] = jnp.pad(c8, (0, 120), constant_values=-1e30)[None, :]


def _tc3_body(u_ref, h1_ref, wres2_ref, l2w_ref, l2b_ref, y_ref):
    pre = (u_ref[0] + u_ref[1]) + _dotT(h1_ref[...], wres2_ref[...])
    t = jnp.where(pre > 0, pre, jnp.exp(jnp.minimum(pre, 0.0)) - 1.0)
    nrm = jnp.sqrt(jnp.sum(t * t, axis=1, keepdims=True))
    t = t / jnp.maximum(nrm, 1e-12)
    y_ref[...] = _leaky(_dotT(t, l2w_ref[...]) + l2b_ref[...][None, :], 0.01)


def _sca_body(sd_hbm, ss_hbm, c_hbm, dst_hbm, src_hbm, et_hbm,
              p_hbm, dpart_hbm,
              sd_v, ss_v, c_v, dst_v, src_v, et_v, p_v, zrow_v, dn_sh):
    cid = lax.axis_index("c")
    sid = lax.axis_index("s")
    wid = sid * 2 + cid
    pltpu.sync_copy(sd_hbm, sd_v)
    pltpu.sync_copy(ss_hbm, ss_v)
    pltpu.sync_copy(c_hbm, c_v)
    pltpu.sync_copy(dst_hbm.at[wid], dst_v)
    pltpu.sync_copy(src_hbm.at[wid], src_v)
    pltpu.sync_copy(et_hbm.at[wid], et_v)

    z16 = jnp.zeros((16,), jnp.float32)
    for g in range(8):
        zrow_v[pl.ds(g * 16, 16)] = z16

    def zbody(k, car):
        pltpu.sync_copy(zrow_v, dn_sh.at[pl.ds((sid * 5 + k) * 128, 128)])
        return car
    lax.fori_loop(0, 5, zbody, 0)
    plsc.subcore_barrier()

    def mbody(i, m):
        return jnp.maximum(m, ss_v[pl.ds(i * 16, 16)])
    m16 = lax.fori_loop(0, N // 16, mbody, jnp.full((16,), -1e30, jnp.float32))

    gdn = lax.GatherDimensionNumbers(offset_dims=(), collapsed_slice_dims=(0,),
                                     start_index_map=(0,))

    def bcast_max(v):
        iot = lax.iota(jnp.int32, 16)
        for sft in (8, 4, 2, 1):
            perm = jnp.bitwise_and(iot + sft, 15)
            pv = lax.gather(v, perm[:, None], gdn, slice_sizes=(1,),
                            mode=lax.GatherScatterMode.PROMISE_IN_BOUNDS)
            v = jnp.maximum(v, pv)
        return v
    kmax = bcast_max(m16) + bcast_max(c_v[...])

    lane = lax.iota(jnp.int32, 16)

    def chunk(j, car):
        for g in range(8):
            sl = pl.ds(g * 16, 16)
            d16 = dst_v[j, sl]
            s16 = src_v[j, sl]
            t16 = et_v[j, sl]
            sdg = plsc.load_gather(sd_v, [d16])
            ssg = plsc.load_gather(ss_v, [s16])
            cg = plsc.load_gather(c_v, [t16])
            z = sdg + ssg + cg
            e = jnp.where(z > 0, z, 0.2 * z)
            zm = sdg + kmax
            mub = jnp.where(zm > 0, zm, 0.2 * zm)
            p = jnp.exp(e - mub)
            valid = (j * 128 + g * 16 + lane) < EW
            p_v[j, sl] = jnp.where(valid, p, 0.0)
        return car
    lax.fori_loop(0, CH, chunk, 0)

    pltpu.sync_copy(p_v, p_hbm.at[wid])

    def scat(j, car):
        pltpu.sync_copy(p_v.at[j], dn_sh.at[dst_v.at[j]], add=True)
        return car
    lax.fori_loop(0, CH, scat, 0)
    plsc.subcore_barrier()

    @pl.when(sid == 0)
    def _():
        pltpu.sync_copy(dn_sh, dpart_hbm.at[cid])


def _zero_rows(rows_v):
    z16 = jnp.zeros((16,), jnp.float32)

    def zr(r, car):
        for g in range(4):
            rows_v[r, pl.ds(g * 16, 16)] = z16
        return car
    lax.fori_loop(0, 128, zr, 0)


def _scb_body(pm_hbm, pa_hbm, dm_hbm, da_hbm, coef_hbm, dst_hbm, src_hbm,
              wxl_hbm, wxr_hbm, u_hbm,
              pm_v, pa_v, dst_v, src_v, dpk_v, dnm_v, dna_v, coef_v, rows_v,
              agg_sh):
    cid = lax.axis_index("c")
    sid = lax.axis_index("s")
    wid = sid * 2 + cid
    pltpu.sync_copy(dst_hbm.at[wid], dst_v)
    pltpu.sync_copy(src_hbm.at[wid], src_v)
    pltpu.sync_copy(pm_hbm.at[wid], pm_v)
    pltpu.sync_copy(pa_hbm.at[wid], pa_v)
    pltpu.sync_copy(coef_hbm, coef_v)

    def _sum_denom(d_hbm, dn_v):
        pltpu.sync_copy(d_hbm.at[0], dn_v)
        pltpu.sync_copy(d_hbm.at[1], dpk_v)

        def sbody(i, car):
            sl = pl.ds(i * 16, 16)
            d = dn_v[sl] + dpk_v[sl] + 1e-16
            r = plsc.bitcast(jnp.full((16,), 0x7EF311C3, jnp.int32)
                             - plsc.bitcast(d, jnp.int32), jnp.float32)
            r = r * (2.0 - d * r)
            r = r * (2.0 - d * r)
            r = r * (2.0 - d * r)
            dn_v[sl] = r
            return car
        lax.fori_loop(0, NP // 16, sbody, 0)
    _sum_denom(dm_hbm, dnm_v)
    _sum_denom(da_hbm, dna_v)

    cm = coef_v[0, :]
    ca = coef_v[1, :]

    def wchunk(j, car):
        for g in range(8):
            sl = pl.ds(g * 16, 16)
            d16 = dst_v[j, sl]
            dnm_g = plsc.load_gather(dnm_v, [d16])
            dna_g = plsc.load_gather(dna_v, [d16])
            w = cm * pm_v[j, sl] * dnm_g + ca * pa_v[j, sl] * dna_g
            pm_v[j, sl] = w
        return car
    lax.fori_loop(0, CH, wchunk, 0)

    for half, wx_hbm in ((0, wxl_hbm), (1, wxr_hbm)):
        _zero_rows(rows_v)

        def za(k, car):
            pltpu.sync_copy(rows_v, agg_sh.at[pl.ds((sid * 5 + k) * 128, 128)])
            return car
        lax.fori_loop(0, 5, za, 0)
        plsc.subcore_barrier()

        gdn = lax.GatherDimensionNumbers(offset_dims=(),
                                         collapsed_slice_dims=(0,),
                                         start_index_map=(0,))

        def chunk(j, car):
            pltpu.sync_copy(wx_hbm.at[src_v.at[j]], rows_v)
            for g in range(8):
                w16 = pm_v[j, pl.ds(g * 16, 16)]
                for r in range(16):
                    wb = lax.gather(w16, jnp.full((16, 1), r, jnp.int32), gdn,
                                    slice_sizes=(1,),
                                    mode=lax.GatherScatterMode.PROMISE_IN_BOUNDS)
                    row = g * 16 + r
                    for q in range(4):
                        sl = pl.ds(q * 16, 16)
                        rows_v[row, sl] = rows_v[row, sl] * wb
            pltpu.sync_copy(rows_v, agg_sh.at[dst_v.at[j]], add=True)
            return car
        lax.fori_loop(0, CH, chunk, 0)
        plsc.subcore_barrier()
        rows = NP // 16
        pltpu.sync_copy(agg_sh.at[pl.ds(sid * rows, rows)],
                        u_hbm.at[half, cid, pl.ds(sid * rows, rows)])
        plsc.subcore_barrier()


def _full(shape):
    return pl.BlockSpec(shape, lambda i: tuple(0 for _ in shape))


def kernel(x, edge_index, edge_type, lin1_w, lin1_b, lin2_w, lin2_b,
           W1, Wr1, a1, Wres1, rel1, W2, Wr2, a2, Wres2, rel2):
    f32 = jnp.float32
    src = edge_index[0]
    dst = edge_index[1]

    def prep(v):
        v = v.reshape(NW, EW)
        v = jnp.pad(v, ((0, 0), (0, EWP - EW)))
        return v.reshape(NW, CH, 128)
    dstE = prep(dst)
    srcE = prep(src)
    etE = prep(edge_type)

    tc1 = pl.pallas_call(
        _tc1_body,
        grid=(GRID,),
        in_specs=[
            pl.BlockSpec((RB, D), lambda i: (i, 0)),
            _full((D, D)),
            _full((D,)),
            _full((D, D)),
            _full((1, 3 * D)),
            _full((D, 200)),
            _full((8, 200)),
        ],
        out_specs=[
            pl.BlockSpec((RB, D), lambda i: (i, 0)),
            pl.BlockSpec((RB, D), lambda i: (i, 0)),
            pl.BlockSpec((1, 1, RB), lambda i: (i, 0, 0)),
            pl.BlockSpec((1, 1, RB), lambda i: (i, 0, 0)),
            _full((1, 128)),
        ],
        out_shape=[
            jax.ShapeDtypeStruct((N, D), f32),
            jax.ShapeDtypeStruct((N, D), f32),
            jax.ShapeDtypeStruct((GRID, 1, RB), f32),
            jax.ShapeDtypeStruct((GRID, 1, RB), f32),
            jax.ShapeDtypeStruct((1, 128), f32),
        ],
    )
    h0, wx1, sd1, ss1, c1 = tc1(x, lin1_w, lin1_b, W1, a1, Wr1, rel1)

    mesh = plsc.VectorSubcoreMesh(core_axis_name="c", subcore_axis_name="s")

    sca = pl.kernel(
        _sca_body,
        out_type=(jax.ShapeDtypeStruct((NW, CH, 128), f32),
                  jax.ShapeDtypeStruct((2, NP), f32)),
        mesh=mesh,
        compiler_params=pltpu.CompilerParams(needs_layout_passes=False),
        scratch_types=[
            pltpu.VMEM((N,), f32),
            pltpu.VMEM((N,), f32),
            pltpu.VMEM((16,), f32),
            pltpu.VMEM((CH, 128), jnp.int32),
            pltpu.VMEM((CH, 128), jnp.int32),
            pltpu.VMEM((CH, 128), jnp.int32),
            pltpu.VMEM((CH, 128), f32),
            pltpu.VMEM((128,), f32),
            pltpu.VMEM_SHARED((NP,), f32),
        ],
    )
    p1, dpart1 = sca(sd1.reshape(N), ss1.reshape(N), c1[0, :16],
                     dstE, srcE, etE)

    scb = pl.kernel(
        _scb_body,
        out_type=jax.ShapeDtypeStruct((2, 2, NP, 64), f32),
        mesh=mesh,
        compiler_params=pltpu.CompilerParams(needs_layout_passes=False,
                                             use_tc_tiling_on_sc=False),
        scratch_types=[
            pltpu.VMEM((CH, 128), f32),
            pltpu.VMEM((CH, 128), f32),
            pltpu.VMEM((CH, 128), jnp.int32),
            pltpu.VMEM((CH, 128), jnp.int32),
            pltpu.VMEM((NP,), f32),
            pltpu.VMEM((NP,), f32),
            pltpu.VMEM((NP,), f32),
            pltpu.VMEM((2, 16), f32),
            pltpu.VMEM((128, 64), f32),
            pltpu.VMEM_SHARED((NP, 64), f32),
        ],
    )

    def run_scb(pm, pa, dm, da, coef, wx):
        uh = scb(pm, pa, dm, da, coef, dstE, srcE, wx[:, :64], wx[:, 64:])
        return jnp.concatenate([uh[0], uh[1]], axis=-1)[:, :N]

    coef1 = jnp.stack([jnp.full((16,), 1.0, f32), jnp.zeros((16,), f32)])
    u1 = run_scb(p1, p1, dpart1, dpart1, coef1, wx1)

    tc2 = pl.pallas_call(
        _tc2_body,
        grid=(GRID,),
        in_specs=[
            pl.BlockSpec((2, RB, D), lambda i: (0, i, 0)),
            pl.BlockSpec((RB, D), lambda i: (i, 0)),
            _full((D, D)),
            _full((D, D)),
            _full((1, 3 * D)),
            _full((D, 200)),
            _full((8, 200)),
        ],
        out_specs=[
            pl.BlockSpec((RB, D), lambda i: (i, 0)),
            pl.BlockSpec((RB, D), lambda i: (i, 0)),
            pl.BlockSpec((1, 1, RB), lambda i: (i, 0, 0)),
            pl.BlockSpec((1, 1, RB), lambda i: (i, 0, 0)),
            _full((1, 128)),
        ],
        out_shape=[
            jax.ShapeDtypeStruct((N, D), f32),
            jax.ShapeDtypeStruct((N, D), f32),
            jax.ShapeDtypeStruct((GRID, 1, RB), f32),
            jax.ShapeDtypeStruct((GRID, 1, RB), f32),
            jax.ShapeDtypeStruct((1, 128), f32),
        ],
    )
    h1, wx2, sd2, ss2, c2 = tc2(u1, h0, Wres1, W2, a2, Wr2, rel2)

    p2, dpart2 = sca(sd2.reshape(N), ss2.reshape(N), c2[0, :16],
                     dstE, srcE, etE)

    coef2 = jnp.stack([jnp.full((16,), 1.0 - BETA, f32),
                       jnp.full((16,), BETA, f32)])
    u2 = run_scb(p2, p1, dpart2, dpart1, coef2, wx2)

    tc3 = pl.pallas_call(
        _tc3_body,
        grid=(GRID,),
        in_specs=[
            pl.BlockSpec((2, RB, D), lambda i: (0, i, 0)),
            pl.BlockSpec((RB, D), lambda i: (i, 0)),
            _full((D, D)),
            _full((D, D)),
            _full((D,)),
        ],
        out_specs=pl.BlockSpec((RB, D), lambda i: (i, 0)),
        out_shape=jax.ShapeDtypeStruct((N, D), f32),
    )
    return tc3(u2, h1, Wres2, lin2_w, lin2_b)


# speedup vs baseline: 12.2642x; 1.1938x over previous
import jax
import jax.numpy as jnp
from jax import lax
from jax.experimental import pallas as pl
from jax.experimental.pallas import tpu as pltpu
from jax.experimental.pallas import tpu_sc as plsc

N = 10000
D = 128
E = 320000
BETA = 0.05
NW = 32
EW = E // NW
EWP = 10240
CH = EWP // 128
NP = 10240
RB = 2000
GRID = N // RB


def _leaky(x, s):
    return jnp.where(x > 0, x, s * x)


def _dotT(a, b):
    return lax.dot_general(a, b, (((1,), (1,)), ((), ())),
                           preferred_element_type=jnp.float32)


def _tc1_body(x_ref, l1w_ref, l1b_ref, w_ref, a_ref, wr_ref, rel_ref,
              h0_ref, wx_ref, sd_ref, ss_ref, c_ref):
    i = pl.program_id(0)
    h0 = _leaky(_dotT(x_ref[...], l1w_ref[...]) + l1b_ref[...][None, :], 0.01)
    wx = _dotT(h0, w_ref[...])
    a = a_ref[...]
    h0_ref[...] = h0
    wx_ref[...] = wx
    sd_ref[...] = jnp.sum(wx * a[0, 0:D][None, :], axis=1)[None, None, :]
    ss_ref[...] = jnp.sum(wx * a[0, D:2 * D][None, :], axis=1)[None, None, :]

    @pl.when(i == 0)
    def _():
        rw = _dotT(rel_ref[...], wr_ref[...])
        c8 = jnp.sum(rw * a[0, 2 * D:3 * D][None, :], axis=1)
        c_ref[...] = jnp.pad(c8, (0, 120), constant_values=-1e30)[None, :]


def _tc2_body(u_ref, h0_ref, wres_ref, w2_ref, a2_ref, wr2_ref,
              rel2_ref, h1_ref, wx2_ref, sd2_ref, ss2_ref, c2_ref):
    i = pl.program_id(0)
    agg = u_ref[0] + u_ref[1]
    pre = agg + _dotT(h0_ref[...], wres_ref[...])
    h1 = jnp.where(pre > 0, pre, jnp.exp(jnp.minimum(pre, 0.0)) - 1.0)
    wx2 = _dotT(h1, w2_ref[...])
    a2 = a2_ref[...]
    h1_ref[...] = h1
    wx2_ref[...] = wx2
    sd2_ref[...] = jnp.sum(wx2 * a2[0, 0:D][None, :], axis=1)[None, None, :]
    ss2_ref[...] = jnp.sum(wx2 * a2[0, D:2 * D][None, :], axis=1)[None, None, :]

    @pl.when(i == 0)
    def _():
        rw = _dotT(rel2_ref[...], wr2_ref[...])
        c8 = jnp.sum(rw * a2[0, 2 * D:3 * D][None, :], axis=1)
        c2_ref[...] = jnp.pad(c8, (0, 120), constant_values=-1e30)[None, :]


def _tc3_body(u_ref, h1_ref, wres2_ref, l2w_ref, l2b_ref, y_ref):
    pre = (u_ref[0] + u_ref[1]) + _dotT(h1_ref[...], wres2_ref[...])
    t = jnp.where(pre > 0, pre, jnp.exp(jnp.minimum(pre, 0.0)) - 1.0)
    nrm = jnp.sqrt(jnp.sum(t * t, axis=1, keepdims=True))
    t = t / jnp.maximum(nrm, 1e-12)
    y_ref[...] = _leaky(_dotT(t, l2w_ref[...]) + l2b_ref[...][None, :], 0.01)


def _sca_body(sd_hbm, ss_hbm, c_hbm, dst_hbm, src_hbm, et_hbm,
              p_hbm, dpart_hbm,
              sd_v, ss_v, c_v, dst_v, src_v, et_v, p_v, zrow_v, dn_sh):
    cid = lax.axis_index("c")
    sid = lax.axis_index("s")
    wid = sid * 2 + cid
    pltpu.sync_copy(sd_hbm, sd_v)
    pltpu.sync_copy(ss_hbm, ss_v)
    pltpu.sync_copy(c_hbm, c_v)
    pltpu.sync_copy(dst_hbm.at[wid], dst_v)
    pltpu.sync_copy(src_hbm.at[wid], src_v)
    pltpu.sync_copy(et_hbm.at[wid], et_v)

    z16 = jnp.zeros((16,), jnp.float32)
    for g in range(8):
        zrow_v[pl.ds(g * 16, 16)] = z16

    def zbody(k, car):
        pltpu.sync_copy(zrow_v, dn_sh.at[pl.ds((sid * 5 + k) * 128, 128)])
        return car
    lax.fori_loop(0, 5, zbody, 0)
    plsc.subcore_barrier()

    def mbody(i, m):
        return jnp.maximum(m, ss_v[pl.ds(i * 16, 16)])
    m16 = lax.fori_loop(0, N // 16, mbody, jnp.full((16,), -1e30, jnp.float32))

    gdn = lax.GatherDimensionNumbers(offset_dims=(), collapsed_slice_dims=(0,),
                                     start_index_map=(0,))

    def bcast_max(v):
        iot = lax.iota(jnp.int32, 16)
        for sft in (8, 4, 2, 1):
            perm = jnp.bitwise_and(iot + sft, 15)
            pv = lax.gather(v, perm[:, None], gdn, slice_sizes=(1,),
                            mode=lax.GatherScatterMode.PROMISE_IN_BOUNDS)
            v = jnp.maximum(v, pv)
        return v
    kmax = bcast_max(m16) + bcast_max(c_v[...])

    lane = lax.iota(jnp.int32, 16)

    def chunk(j, car):
        for g in range(8):
            sl = pl.ds(g * 16, 16)
            d16 = dst_v[j, sl]
            s16 = src_v[j, sl]
            t16 = et_v[j, sl]
            sdg = plsc.load_gather(sd_v, [d16])
            ssg = plsc.load_gather(ss_v, [s16])
            cg = plsc.load_gather(c_v, [t16])
            z = sdg + ssg + cg
            e = jnp.where(z > 0, z, 0.2 * z)
            zm = sdg + kmax
            mub = jnp.where(zm > 0, zm, 0.2 * zm)
            p = jnp.exp(e - mub)
            valid = (j * 128 + g * 16 + lane) < EW
            p_v[j, sl] = jnp.where(valid, p, 0.0)
        return car
    lax.fori_loop(0, CH, chunk, 0)

    pltpu.sync_copy(p_v, p_hbm.at[wid])

    def scat(j, car):
        pltpu.sync_copy(p_v.at[j], dn_sh.at[dst_v.at[j]], add=True)
        return car
    lax.fori_loop(0, CH, scat, 0)
    plsc.subcore_barrier()

    @pl.when(sid == 0)
    def _():
        pltpu.sync_copy(dn_sh, dpart_hbm.at[cid])


def _zero_rows(rows_v):
    z16 = jnp.zeros((16,), jnp.float32)

    def zr(r, car):
        for g in range(4):
            rows_v[r, pl.ds(g * 16, 16)] = z16
        return car
    lax.fori_loop(0, 128, zr, 0)


def _scb_body(pm_hbm, pa_hbm, dm_hbm, da_hbm, coef_hbm, dst_hbm, src_hbm,
              wxl_hbm, wxr_hbm, u_hbm,
              pm_v, pa_v, dst_v, src_v, dpk_v, dnm_v, dna_v, coef_v, rows_v,
              rows2_v, sem0, sem1, agg_sh):
    cid = lax.axis_index("c")
    sid = lax.axis_index("s")
    wid = sid * 2 + cid
    pltpu.sync_copy(dst_hbm.at[wid], dst_v)
    pltpu.sync_copy(src_hbm.at[wid], src_v)
    pltpu.sync_copy(pm_hbm.at[wid], pm_v)
    pltpu.sync_copy(pa_hbm.at[wid], pa_v)
    pltpu.sync_copy(coef_hbm, coef_v)

    def _sum_denom(d_hbm, dn_v):
        pltpu.sync_copy(d_hbm.at[0], dn_v)
        pltpu.sync_copy(d_hbm.at[1], dpk_v)

        def sbody(i, car):
            sl = pl.ds(i * 16, 16)
            d = dn_v[sl] + dpk_v[sl] + 1e-16
            r = plsc.bitcast(jnp.full((16,), 0x7EF311C3, jnp.int32)
                             - plsc.bitcast(d, jnp.int32), jnp.float32)
            r = r * (2.0 - d * r)
            r = r * (2.0 - d * r)
            r = r * (2.0 - d * r)
            dn_v[sl] = r
            return car
        lax.fori_loop(0, NP // 16, sbody, 0)
    _sum_denom(dm_hbm, dnm_v)
    _sum_denom(da_hbm, dna_v)

    cm = coef_v[0, :]
    ca = coef_v[1, :]

    def wchunk(j, car):
        for g in range(8):
            sl = pl.ds(g * 16, 16)
            d16 = dst_v[j, sl]
            dnm_g = plsc.load_gather(dnm_v, [d16])
            dna_g = plsc.load_gather(dna_v, [d16])
            w = cm * pm_v[j, sl] * dnm_g + ca * pa_v[j, sl] * dna_g
            pm_v[j, sl] = w
        return car
    lax.fori_loop(0, CH, wchunk, 0)

    for half, wx_hbm in ((0, wxl_hbm), (1, wxr_hbm)):
        _zero_rows(rows_v)

        def za(k, car):
            pltpu.sync_copy(rows_v, agg_sh.at[pl.ds((sid * 5 + k) * 128, 128)])
            return car
        lax.fori_loop(0, 5, za, 0)
        plsc.subcore_barrier()

        gdn = lax.GatherDimensionNumbers(offset_dims=(),
                                         collapsed_slice_dims=(0,),
                                         start_index_map=(0,))

        def scale_scatter(j, buf):
            for g in range(8):
                w16 = pm_v[j, pl.ds(g * 16, 16)]
                for r in range(16):
                    wb = lax.gather(w16, jnp.full((16, 1), r, jnp.int32), gdn,
                                    slice_sizes=(1,),
                                    mode=lax.GatherScatterMode.PROMISE_IN_BOUNDS)
                    row = g * 16 + r
                    for q in range(4):
                        sl = pl.ds(q * 16, 16)
                        buf[row, sl] = buf[row, sl] * wb
            pltpu.sync_copy(buf, agg_sh.at[dst_v.at[j]], add=True)

        pltpu.async_copy(wx_hbm.at[src_v.at[0]], rows_v, sem0)

        def pair(i, car):
            j0 = 2 * i
            pltpu.async_copy(wx_hbm.at[src_v.at[j0 + 1]], rows2_v, sem1)
            pltpu.make_async_copy(wx_hbm.at[src_v.at[j0]], rows_v,
                                  sem0).wait()
            scale_scatter(j0, rows_v)

            @pl.when(j0 + 2 < CH)
            def _():
                pltpu.async_copy(wx_hbm.at[src_v.at[j0 + 2]], rows_v, sem0)
            pltpu.make_async_copy(wx_hbm.at[src_v.at[j0 + 1]], rows2_v,
                                  sem1).wait()
            scale_scatter(j0 + 1, rows2_v)
            return car
        lax.fori_loop(0, CH // 2, pair, 0)
        plsc.subcore_barrier()
        rows = NP // 16
        pltpu.sync_copy(agg_sh.at[pl.ds(sid * rows, rows)],
                        u_hbm.at[half, cid, pl.ds(sid * rows, rows)])
        plsc.subcore_barrier()


def _full(shape):
    return pl.BlockSpec(shape, lambda i: tuple(0 for _ in shape))


def kernel(x, edge_index, edge_type, lin1_w, lin1_b, lin2_w, lin2_b,
           W1, Wr1, a1, Wres1, rel1, W2, Wr2, a2, Wres2, rel2):
    f32 = jnp.float32
    src = edge_index[0]
    dst = edge_index[1]

    def prep(v):
        v = v.reshape(NW, EW)
        v = jnp.pad(v, ((0, 0), (0, EWP - EW)))
        return v.reshape(NW, CH, 128)
    dstE = prep(dst)
    srcE = prep(src)
    etE = prep(edge_type)

    tc1 = pl.pallas_call(
        _tc1_body,
        grid=(GRID,),
        in_specs=[
            pl.BlockSpec((RB, D), lambda i: (i, 0)),
            _full((D, D)),
            _full((D,)),
            _full((D, D)),
            _full((1, 3 * D)),
            _full((D, 200)),
            _full((8, 200)),
        ],
        out_specs=[
            pl.BlockSpec((RB, D), lambda i: (i, 0)),
            pl.BlockSpec((RB, D), lambda i: (i, 0)),
            pl.BlockSpec((1, 1, RB), lambda i: (i, 0, 0)),
            pl.BlockSpec((1, 1, RB), lambda i: (i, 0, 0)),
            _full((1, 128)),
        ],
        out_shape=[
            jax.ShapeDtypeStruct((N, D), f32),
            jax.ShapeDtypeStruct((N, D), f32),
            jax.ShapeDtypeStruct((GRID, 1, RB), f32),
            jax.ShapeDtypeStruct((GRID, 1, RB), f32),
            jax.ShapeDtypeStruct((1, 128), f32),
        ],
    )
    h0, wx1, sd1, ss1, c1 = tc1(x, lin1_w, lin1_b, W1, a1, Wr1, rel1)

    mesh = plsc.VectorSubcoreMesh(core_axis_name="c", subcore_axis_name="s")

    sca = pl.kernel(
        _sca_body,
        out_type=(jax.ShapeDtypeStruct((NW, CH, 128), f32),
                  jax.ShapeDtypeStruct((2, NP), f32)),
        mesh=mesh,
        compiler_params=pltpu.CompilerParams(needs_layout_passes=False),
        scratch_types=[
            pltpu.VMEM((N,), f32),
            pltpu.VMEM((N,), f32),
            pltpu.VMEM((16,), f32),
            pltpu.VMEM((CH, 128), jnp.int32),
            pltpu.VMEM((CH, 128), jnp.int32),
            pltpu.VMEM((CH, 128), jnp.int32),
            pltpu.VMEM((CH, 128), f32),
            pltpu.VMEM((128,), f32),
            pltpu.VMEM_SHARED((NP,), f32),
        ],
    )
    p1, dpart1 = sca(sd1.reshape(N), ss1.reshape(N), c1[0, :16],
                     dstE, srcE, etE)

    scb = pl.kernel(
        _scb_body,
        out_type=jax.ShapeDtypeStruct((2, 2, NP, 64), f32),
        mesh=mesh,
        compiler_params=pltpu.CompilerParams(needs_layout_passes=False,
                                             use_tc_tiling_on_sc=False),
        scratch_types=[
            pltpu.VMEM((CH, 128), f32),
            pltpu.VMEM((CH, 128), f32),
            pltpu.VMEM((CH, 128), jnp.int32),
            pltpu.VMEM((CH, 128), jnp.int32),
            pltpu.VMEM((NP,), f32),
            pltpu.VMEM((NP,), f32),
            pltpu.VMEM((NP,), f32),
            pltpu.VMEM((2, 16), f32),
            pltpu.VMEM((128, 64), f32),
            pltpu.VMEM((128, 64), f32),
            pltpu.SemaphoreType.DMA,
            pltpu.SemaphoreType.DMA,
            pltpu.VMEM_SHARED((NP, 64), f32),
        ],
    )

    def run_scb(pm, pa, dm, da, coef, wx):
        uh = scb(pm, pa, dm, da, coef, dstE, srcE, wx[:, :64], wx[:, 64:])
        return jnp.concatenate([uh[0], uh[1]], axis=-1)[:, :N]

    coef1 = jnp.stack([jnp.full((16,), 1.0, f32), jnp.zeros((16,), f32)])
    u1 = run_scb(p1, p1, dpart1, dpart1, coef1, wx1)

    tc2 = pl.pallas_call(
        _tc2_body,
        grid=(GRID,),
        in_specs=[
            pl.BlockSpec((2, RB, D), lambda i: (0, i, 0)),
            pl.BlockSpec((RB, D), lambda i: (i, 0)),
            _full((D, D)),
            _full((D, D)),
            _full((1, 3 * D)),
            _full((D, 200)),
            _full((8, 200)),
        ],
        out_specs=[
            pl.BlockSpec((RB, D), lambda i: (i, 0)),
            pl.BlockSpec((RB, D), lambda i: (i, 0)),
            pl.BlockSpec((1, 1, RB), lambda i: (i, 0, 0)),
            pl.BlockSpec((1, 1, RB), lambda i: (i, 0, 0)),
            _full((1, 128)),
        ],
        out_shape=[
            jax.ShapeDtypeStruct((N, D), f32),
            jax.ShapeDtypeStruct((N, D), f32),
            jax.ShapeDtypeStruct((GRID, 1, RB), f32),
            jax.ShapeDtypeStruct((GRID, 1, RB), f32),
            jax.ShapeDtypeStruct((1, 128), f32),
        ],
    )
    h1, wx2, sd2, ss2, c2 = tc2(u1, h0, Wres1, W2, a2, Wr2, rel2)

    p2, dpart2 = sca(sd2.reshape(N), ss2.reshape(N), c2[0, :16],
                     dstE, srcE, etE)

    coef2 = jnp.stack([jnp.full((16,), 1.0 - BETA, f32),
                       jnp.full((16,), BETA, f32)])
    u2 = run_scb(p2, p1, dpart2, dpart1, coef2, wx2)

    tc3 = pl.pallas_call(
        _tc3_body,
        grid=(GRID,),
        in_specs=[
            pl.BlockSpec((2, RB, D), lambda i: (0, i, 0)),
            pl.BlockSpec((RB, D), lambda i: (i, 0)),
            _full((D, D)),
            _full((D, D)),
            _full((D,)),
        ],
        out_specs=pl.BlockSpec((RB, D), lambda i: (i, 0)),
        out_shape=jax.ShapeDtypeStruct((N, D), f32),
    )
    return tc3(u2, h1, Wres2, lin2_w, lin2_b)
